# Initial kernel scaffold; baseline (speedup 1.0000x reference)
import functools

import jax
import jax.numpy as jnp
from jax import lax
from jax.experimental import pallas as pl
from jax.experimental.pallas import tpu as pltpu
from jax.experimental.pallas import tpu_sc as plsc

N = 100000
E = 6400000
CH = 128
NCH = E // CH
NW = 32
ROWS_PER_TILE = 6272
N_PAD = 16 * ROWS_PER_TILE
IDCH = ROWS_PER_TILE // CH
QUOT, REM = divmod(NCH, NW)

_mesh = plsc.VectorSubcoreMesh(core_axis_name="c", subcore_axis_name="s")


def _pass_body(first, u_in, src2d, dst2d, zeros, ident, out,
               u_sp, acc_sp, srcb, dstb, rowsb, idb, bb):
    c = lax.axis_index("c")
    s = lax.axis_index("s")
    r0 = s * ROWS_PER_TILE
    if first:
        pltpu.sync_copy(u_in.at[pl.ds(r0, ROWS_PER_TILE)],
                        u_sp.at[pl.ds(r0, ROWS_PER_TILE)])
    else:
        pltpu.sync_copy(u_in.at[0, pl.ds(r0, ROWS_PER_TILE)],
                        u_sp.at[pl.ds(r0, ROWS_PER_TILE)])
        pltpu.sync_copy(u_in.at[1, pl.ds(r0, ROWS_PER_TILE)], bb)
        pltpu.sync_copy(ident.at[pl.ds(s * IDCH, IDCH)], idb)

        def add_body(j, carry):
            pltpu.sync_copy(bb.at[pl.ds(j * CH, CH)],
                            u_sp.at[idb.at[j]], add=True)
            return carry
        lax.fori_loop(0, IDCH, add_body, 0)
    pltpu.sync_copy(zeros.at[pl.ds(r0, ROWS_PER_TILE)],
                    acc_sp.at[pl.ds(r0, ROWS_PER_TILE)])
    plsc.subcore_barrier()

    gwid = c * 16 + s
    nrows = QUOT + (gwid < REM).astype(jnp.int32)
    base = gwid * QUOT + jnp.minimum(gwid, REM)

    def chunk_body(i, carry):
        row = base + i
        pltpu.sync_copy(src2d.at[pl.ds(row, 1)], srcb)
        pltpu.sync_copy(dst2d.at[pl.ds(row, 1)], dstb)
        pltpu.sync_copy(u_sp.at[srcb.at[0]], rowsb.at[0])
        pltpu.sync_copy(rowsb.at[0], acc_sp.at[dstb.at[0]], add=True)
        return carry
    lax.fori_loop(0, nrows, chunk_body, 0)
    plsc.subcore_barrier()
    pltpu.sync_copy(acc_sp.at[pl.ds(r0, ROWS_PER_TILE)],
                    out.at[c, pl.ds(r0, ROWS_PER_TILE)])


def _make_pass(first):
    in_shape = ((N_PAD, 4) if first else (2, N_PAD, 4))
    return pl.kernel(
        functools.partial(_pass_body, first),
        out_type=jax.ShapeDtypeStruct((2, N_PAD, 4), jnp.float32),
        mesh=_mesh,
        scratch_types=[
            pltpu.VMEM_SHARED((N_PAD, 4), jnp.float32),
            pltpu.VMEM_SHARED((N_PAD, 4), jnp.float32),
            pltpu.VMEM((1, CH), jnp.int32),
            pltpu.VMEM((1, CH), jnp.int32),
            pltpu.VMEM((1, CH, 4), jnp.float32),
            pltpu.VMEM((IDCH, CH), jnp.int32),
            pltpu.VMEM((ROWS_PER_TILE, 4), jnp.float32),
        ],
    )


_pass_first = _make_pass(True)
_pass_next = _make_pass(False)


def _prologue_body(x_ref, z_ref, m_ref):
    x = x_ref[...]
    m = jnp.mean(x, axis=0, keepdims=True)
    xc = x - m
    z = jnp.concatenate([xc, jnp.ones((N, 1), jnp.float32)], axis=1)
    zpad = jnp.concatenate([z, jnp.zeros((N_PAD - N, 4), jnp.float32)], axis=0)
    z_ref[...] = zpad
    m_ref[...] = m


def _prologue(x):
    return pl.pallas_call(
        _prologue_body,
        out_shape=[jax.ShapeDtypeStruct((N_PAD, 4), jnp.float32),
                   jax.ShapeDtypeStruct((1, 3), jnp.float32)],
    )(x)


_BLK = 4000


def _combine_body(p1_ref, p2_ref, p3_ref, x_ref, cst_ref, o_ref):
    pid = pl.program_id(0)
    u1 = p1_ref[0] + p1_ref[1]
    u2 = p2_ref[0] + p2_ref[1]
    u3 = p3_ref[0] + p3_ref[1]
    cst = cst_ref[...]
    M2, M1, M0 = cst[0:3], cst[3:6], cst[6:9]
    v2, v1, cr = cst[9:10], cst[10:11], cst[11:12]
    x3 = (jnp.dot(u3[:, :3], M2) + jnp.dot(u2[:, :3], M1)
          + jnp.dot(u1[:, :3], M0)
          + u2[:, 3:4] * v2 + u1[:, 3:4] * v1 + cr)
    r = pid * _BLK + lax.broadcasted_iota(jnp.int32, (_BLK, 1), 0)
    rm = r % 40
    fixed = (r < 1960) & ((rm <= 13) | ((rm >= 25) & (rm <= 38)))
    o_ref[...] = jnp.where(fixed, x_ref[...], x3)


def _combine(p1, p2, p3, x, cst):
    grid = N // _BLK
    return pl.pallas_call(
        _combine_body,
        grid=(grid,),
        in_specs=[
            pl.BlockSpec((2, _BLK, 4), lambda i: (0, i, 0)),
            pl.BlockSpec((2, _BLK, 4), lambda i: (0, i, 0)),
            pl.BlockSpec((2, _BLK, 4), lambda i: (0, i, 0)),
            pl.BlockSpec((_BLK, 3), lambda i: (i, 0)),
            pl.BlockSpec((12, 3), lambda i: (0, 0)),
        ],
        out_specs=pl.BlockSpec((_BLK, 3), lambda i: (i, 0)),
        out_shape=jax.ShapeDtypeStruct((N, 3), jnp.float32),
    )(p1, p2, p3, x, cst)


def kernel(x, edge_index, edge_weight,
           W1a, b1a, W2a, b2a, W1b, b1b, W2b, b2b, W1c, b1c, W2c, b2c):
    src2d = edge_index[0].reshape(NCH, CH)
    dst2d = edge_index[1].reshape(NCH, CH)
    zeros = jnp.zeros((N_PAD, 4), jnp.float32)
    ident = jnp.arange(N_PAD, dtype=jnp.int32).reshape(N_PAD // CH, CH)

    z, mrow = _prologue(x)
    p1 = _pass_first(z, src2d, dst2d, zeros, ident)
    p2 = _pass_next(p1, src2d, dst2d, zeros, ident)
    p3 = _pass_next(p2, src2d, dst2d, zeros, ident)

    W2b_top, W2b_bot = W2b[:50], W2b[50:]
    Wc1, Wc2, Wc3 = W2c[:50], W2c[50:100], W2c[100:]
    M2 = W2a @ W2b_top @ Wc1
    M1 = W2b_bot @ Wc1 + W2a @ Wc2
    M0 = Wc3
    v2 = (b2a @ W2b_top) @ Wc1
    v1 = b2b @ Wc1 + b2a @ Wc2
    cr = b2c + mrow[0]
    cst = jnp.concatenate(
        [M2, M1, M0, v2[None], v1[None], cr[None]], axis=0)

    return _combine(p1, p2, p3, x, cst)


# speedup vs baseline: 5.5965x; 5.5965x over previous
import functools

import jax
import jax.numpy as jnp
from jax import lax
from jax.experimental import pallas as pl
from jax.experimental.pallas import tpu as pltpu
from jax.experimental.pallas import tpu_sc as plsc

N = 100000
E = 6400000
CH = 128
NCH = E // CH
NW = 32
ROWS_PER_TILE = 6272
N_PAD = 16 * ROWS_PER_TILE
QUOT, REM = divmod(NCH, NW)

_mesh = plsc.VectorSubcoreMesh(core_axis_name="c", subcore_axis_name="s")


def _pass_body(u_in, src1d, dst1d, zeros1, out,
               a0, a1, a2, a3, srcb, dstb, valsb, sem):
    c = lax.axis_index("c")
    s = lax.axis_index("s")
    r0 = s * ROWS_PER_TILE
    accs = (a0, a1, a2, a3)
    for a in accs:
        pltpu.sync_copy(zeros1.at[pl.ds(r0, ROWS_PER_TILE)],
                        a.at[pl.ds(r0, ROWS_PER_TILE)])
    plsc.subcore_barrier()

    gwid = c * 16 + s
    nchunks = QUOT + (gwid < REM).astype(jnp.int32)
    base = gwid * QUOT + jnp.minimum(gwid, REM)

    def chunk_body(i, carry):
        off = (base + i) * CH
        pltpu.sync_copy(src1d.at[pl.ds(off, CH)], srcb)
        pltpu.sync_copy(dst1d.at[pl.ds(off, CH)], dstb)
        for k, a in enumerate(accs):
            pltpu.async_copy(u_in.at[k].at[srcb], valsb, sem).wait()
            pltpu.sync_copy(valsb, a.at[dstb], add=True)
        return carry
    lax.fori_loop(0, nchunks, chunk_body, 0)
    plsc.subcore_barrier()
    for k, a in enumerate(accs):
        pltpu.sync_copy(a.at[pl.ds(r0, ROWS_PER_TILE)],
                        out.at[c, k, pl.ds(r0, ROWS_PER_TILE)])


_sc_pass = pl.kernel(
    _pass_body,
    out_type=jax.ShapeDtypeStruct((2, 4, N_PAD), jnp.float32),
    mesh=_mesh,
    scratch_types=[
        pltpu.VMEM_SHARED((N_PAD,), jnp.float32),
        pltpu.VMEM_SHARED((N_PAD,), jnp.float32),
        pltpu.VMEM_SHARED((N_PAD,), jnp.float32),
        pltpu.VMEM_SHARED((N_PAD,), jnp.float32),
        pltpu.VMEM((CH,), jnp.int32),
        pltpu.VMEM((CH,), jnp.int32),
        pltpu.VMEM((CH,), jnp.float32),
        pltpu.SemaphoreType.DMA,
    ],
    compiler_params=pltpu.CompilerParams(use_tc_tiling_on_sc=False),
)

_PBLK = 3584


def _prologue_body(xT_ref, z_ref, m_ref):
    p = pl.program_id(0)
    i = pl.program_id(1)

    @pl.when((p == 0) & (i == 0))
    def _():
        m_ref[...] = jnp.zeros((3, 1), jnp.float32)

    @pl.when(p == 0)
    def _():
        m_ref[...] += jnp.sum(xT_ref[...], axis=1, keepdims=True) / N

    @pl.when(p == 1)
    def _():
        xc = xT_ref[...] - m_ref[...]
        z_ref[...] = jnp.concatenate(
            [xc, jnp.ones((1, _PBLK), jnp.float32)], axis=0)


def _prologue(xT):
    return pl.pallas_call(
        _prologue_body,
        grid=(2, N_PAD // _PBLK),
        in_specs=[pl.BlockSpec((3, _PBLK), lambda p, i: (0, i))],
        out_specs=[pl.BlockSpec((4, _PBLK), lambda p, i: (0, i)),
                   pl.BlockSpec((3, 1), lambda p, i: (0, 0))],
        out_shape=[jax.ShapeDtypeStruct((4, N_PAD), jnp.float32),
                   jax.ShapeDtypeStruct((3, 1), jnp.float32)],
    )(xT)


def _add_body(p_ref, o_ref):
    o_ref[...] = p_ref[0] + p_ref[1]


def _tc_add(p):
    return pl.pallas_call(
        _add_body,
        grid=(N_PAD // _PBLK,),
        in_specs=[pl.BlockSpec((2, 4, _PBLK), lambda i: (0, 0, i))],
        out_specs=pl.BlockSpec((4, _PBLK), lambda i: (0, i)),
        out_shape=jax.ShapeDtypeStruct((4, N_PAD), jnp.float32),
    )(p)


_BLK = 3584


def _combine_body(u1_ref, u2_ref, u3_ref, xT_ref, cst_ref, o_ref):
    pid = pl.program_id(0)
    u1 = u1_ref[...]
    u2 = u2_ref[...]
    u3 = u3_ref[...]
    cst = cst_ref[...]
    M2T, M1T, M0T = cst[:, 0:3], cst[:, 3:6], cst[:, 6:9]
    v2c, v1c, crc = cst[:, 9:10], cst[:, 10:11], cst[:, 11:12]
    x3 = (jnp.dot(M2T, u3[:3]) + jnp.dot(M1T, u2[:3])
          + jnp.dot(M0T, u1[:3])
          + v2c * u2[3:4] + v1c * u1[3:4] + crc)
    r = pid * _BLK + lax.broadcasted_iota(jnp.int32, (1, _BLK), 1)
    rm = r % 40
    fixed = (r < 1960) & ((rm <= 13) | ((rm >= 25) & (rm <= 38)))
    o_ref[...] = jnp.where(fixed, xT_ref[...], x3)


def _combine(u1, u2, u3, xT, cstT):
    return pl.pallas_call(
        _combine_body,
        grid=(N_PAD // _BLK,),
        in_specs=[
            pl.BlockSpec((4, _BLK), lambda i: (0, i)),
            pl.BlockSpec((4, _BLK), lambda i: (0, i)),
            pl.BlockSpec((4, _BLK), lambda i: (0, i)),
            pl.BlockSpec((3, _BLK), lambda i: (0, i)),
            pl.BlockSpec((3, 12), lambda i: (0, 0)),
        ],
        out_specs=pl.BlockSpec((3, _BLK), lambda i: (0, i)),
        out_shape=jax.ShapeDtypeStruct((3, N_PAD), jnp.float32),
    )(u1, u2, u3, xT, cstT)


def kernel(x, edge_index, edge_weight,
           W1a, b1a, W2a, b2a, W1b, b1b, W2b, b2b, W1c, b1c, W2c, b2c):
    src1d = edge_index[0]
    dst1d = edge_index[1]
    zeros1 = jnp.zeros((N_PAD,), jnp.float32)

    xT = jnp.concatenate(
        [x.T, jnp.zeros((3, N_PAD - N), jnp.float32)], axis=1)
    z, mcol = _prologue(xT)

    u1 = _tc_add(_sc_pass(z, src1d, dst1d, zeros1))
    u2 = _tc_add(_sc_pass(u1, src1d, dst1d, zeros1))
    u3 = _tc_add(_sc_pass(u2, src1d, dst1d, zeros1))

    W2b_top, W2b_bot = W2b[:50], W2b[50:]
    Wc1, Wc2, Wc3 = W2c[:50], W2c[50:100], W2c[100:]
    M2 = W2a @ W2b_top @ Wc1
    M1 = W2b_bot @ Wc1 + W2a @ Wc2
    M0 = Wc3
    v2 = (b2a @ W2b_top) @ Wc1
    v1 = b2b @ Wc1 + b2a @ Wc2
    cr = b2c + mcol[:, 0]
    cstT = jnp.concatenate(
        [M2.T, M1.T, M0.T, v2[:, None], v1[:, None], cr[:, None]], axis=1)

    outT = _combine(u1, u2, u3, xT, cstT)
    return outT[:, :N].T


# speedup vs baseline: 37.1664x; 6.6410x over previous
import functools

import jax
import jax.numpy as jnp
from jax import lax
from jax.experimental import pallas as pl
from jax.experimental.pallas import tpu as pltpu
from jax.experimental.pallas import tpu_sc as plsc

N = 100000
E = 6400000
CH = 128
NCH = E // CH
NW = 32
ROWS_PER_TILE = 6272
N_PAD = 16 * ROWS_PER_TILE
QUOT, REM = divmod(NCH, NW)
BLK_CH = 64
NBLOCKS = (QUOT + 1 + BLK_CH - 1) // BLK_CH

_mesh = plsc.VectorSubcoreMesh(core_axis_name="c", subcore_axis_name="s")


def _pass_body(u_in, src2d, dst2d, zeros1, out,
               a0, a1, a2, a3, u0, u1, u2, u3, sblk, dblk, vals, gsem, ssem):
    c = lax.axis_index("c")
    s = lax.axis_index("s")
    r0 = s * ROWS_PER_TILE
    accs = (a0, a1, a2, a3)
    usp = (u0, u1, u2, u3)
    for a in accs:
        pltpu.sync_copy(zeros1.at[pl.ds(r0, ROWS_PER_TILE)],
                        a.at[pl.ds(r0, ROWS_PER_TILE)])
    for k in range(4):
        pltpu.sync_copy(u_in.at[k].at[pl.ds(r0, ROWS_PER_TILE)],
                        usp[k].at[pl.ds(r0, ROWS_PER_TILE)])
    plsc.subcore_barrier()

    gwid = c * 16 + s
    nchunks = QUOT + (gwid < REM).astype(jnp.int32)
    base = gwid * QUOT + jnp.minimum(gwid, REM)

    def block_body(b, carry):
        row0 = base + b * BLK_CH
        cnt = jnp.minimum(nchunks - b * BLK_CH, BLK_CH)
        pltpu.sync_copy(src2d.at[pl.ds(row0, BLK_CH)], sblk)
        pltpu.sync_copy(dst2d.at[pl.ds(row0, BLK_CH)], dblk)

        def chunk_body(j, carry2):
            descs = [pltpu.async_copy(usp[k].at[sblk.at[j]],
                                      vals.at[k], gsem)
                     for k in range(4)]
            for d in descs:
                d.wait()
            sdescs = [pltpu.async_copy(vals.at[k], accs[k].at[dblk.at[j]],
                                       ssem, add=True)
                      for k in range(4)]
            for d in sdescs:
                d.wait()
            return carry2
        lax.fori_loop(0, cnt, chunk_body, 0)
        return carry
    lax.fori_loop(0, NBLOCKS, block_body, 0)
    plsc.subcore_barrier()
    for k, a in enumerate(accs):
        pltpu.sync_copy(a.at[pl.ds(r0, ROWS_PER_TILE)],
                        out.at[c, k, pl.ds(r0, ROWS_PER_TILE)])


_sc_pass = pl.kernel(
    _pass_body,
    out_type=jax.ShapeDtypeStruct((2, 4, N_PAD), jnp.float32),
    mesh=_mesh,
    scratch_types=[
        pltpu.VMEM_SHARED((N_PAD,), jnp.float32),
        pltpu.VMEM_SHARED((N_PAD,), jnp.float32),
        pltpu.VMEM_SHARED((N_PAD,), jnp.float32),
        pltpu.VMEM_SHARED((N_PAD,), jnp.float32),
        pltpu.VMEM_SHARED((N_PAD,), jnp.float32),
        pltpu.VMEM_SHARED((N_PAD,), jnp.float32),
        pltpu.VMEM_SHARED((N_PAD,), jnp.float32),
        pltpu.VMEM_SHARED((N_PAD,), jnp.float32),
        pltpu.VMEM((BLK_CH, CH), jnp.int32),
        pltpu.VMEM((BLK_CH, CH), jnp.int32),
        pltpu.VMEM((4, CH), jnp.float32),
        pltpu.SemaphoreType.DMA,
        pltpu.SemaphoreType.DMA,
    ],
    compiler_params=pltpu.CompilerParams(use_tc_tiling_on_sc=False),
)

_PBLK = 3584


def _prologue_body(xT_ref, z_ref, m_ref):
    p = pl.program_id(0)
    i = pl.program_id(1)

    @pl.when((p == 0) & (i == 0))
    def _():
        m_ref[...] = jnp.zeros((3, 1), jnp.float32)

    @pl.when(p == 0)
    def _():
        m_ref[...] += jnp.sum(xT_ref[...], axis=1, keepdims=True) / N

    @pl.when(p == 1)
    def _():
        xc = xT_ref[...] - m_ref[...]
        z_ref[...] = jnp.concatenate(
            [xc, jnp.ones((1, _PBLK), jnp.float32)], axis=0)


def _prologue(xT):
    return pl.pallas_call(
        _prologue_body,
        grid=(2, N_PAD // _PBLK),
        in_specs=[pl.BlockSpec((3, _PBLK), lambda p, i: (0, i))],
        out_specs=[pl.BlockSpec((4, _PBLK), lambda p, i: (0, i)),
                   pl.BlockSpec((3, 1), lambda p, i: (0, 0))],
        out_shape=[jax.ShapeDtypeStruct((4, N_PAD), jnp.float32),
                   jax.ShapeDtypeStruct((3, 1), jnp.float32)],
    )(xT)


def _add_body(p_ref, o_ref):
    o_ref[...] = p_ref[0] + p_ref[1]


def _tc_add(p):
    return pl.pallas_call(
        _add_body,
        grid=(N_PAD // _PBLK,),
        in_specs=[pl.BlockSpec((2, 4, _PBLK), lambda i: (0, 0, i))],
        out_specs=pl.BlockSpec((4, _PBLK), lambda i: (0, i)),
        out_shape=jax.ShapeDtypeStruct((4, N_PAD), jnp.float32),
    )(p)


_BLK = 3584


def _combine_body(u1_ref, u2_ref, u3_ref, xT_ref, cst_ref, o_ref):
    pid = pl.program_id(0)
    u1 = u1_ref[...]
    u2 = u2_ref[...]
    u3 = u3_ref[...]
    cst = cst_ref[...]
    M2T, M1T, M0T = cst[:, 0:3], cst[:, 3:6], cst[:, 6:9]
    v2c, v1c, crc = cst[:, 9:10], cst[:, 10:11], cst[:, 11:12]
    x3 = (jnp.dot(M2T, u3[:3]) + jnp.dot(M1T, u2[:3])
          + jnp.dot(M0T, u1[:3])
          + v2c * u2[3:4] + v1c * u1[3:4] + crc)
    r = pid * _BLK + lax.broadcasted_iota(jnp.int32, (1, _BLK), 1)
    rm = r % 40
    fixed = (r < 1960) & ((rm <= 13) | ((rm >= 25) & (rm <= 38)))
    o_ref[...] = jnp.where(fixed, xT_ref[...], x3)


def _combine(u1, u2, u3, xT, cstT):
    return pl.pallas_call(
        _combine_body,
        grid=(N_PAD // _BLK,),
        in_specs=[
            pl.BlockSpec((4, _BLK), lambda i: (0, i)),
            pl.BlockSpec((4, _BLK), lambda i: (0, i)),
            pl.BlockSpec((4, _BLK), lambda i: (0, i)),
            pl.BlockSpec((3, _BLK), lambda i: (0, i)),
            pl.BlockSpec((3, 12), lambda i: (0, 0)),
        ],
        out_specs=pl.BlockSpec((3, _BLK), lambda i: (0, i)),
        out_shape=jax.ShapeDtypeStruct((3, N_PAD), jnp.float32),
    )(u1, u2, u3, xT, cstT)


def kernel(x, edge_index, edge_weight,
           W1a, b1a, W2a, b2a, W1b, b1b, W2b, b2b, W1c, b1c, W2c, b2c):
    pad = jnp.zeros((BLK_CH, CH), jnp.int32)
    src2d = jnp.concatenate([edge_index[0].reshape(NCH, CH), pad], axis=0)
    dst2d = jnp.concatenate([edge_index[1].reshape(NCH, CH), pad], axis=0)
    zeros1 = jnp.zeros((N_PAD,), jnp.float32)

    xT = jnp.concatenate(
        [x.T, jnp.zeros((3, N_PAD - N), jnp.float32)], axis=1)
    z, mcol = _prologue(xT)

    u1 = _tc_add(_sc_pass(z, src2d, dst2d, zeros1))
    u2 = _tc_add(_sc_pass(u1, src2d, dst2d, zeros1))
    u3 = _tc_add(_sc_pass(u2, src2d, dst2d, zeros1))

    W2b_top, W2b_bot = W2b[:50], W2b[50:]
    Wc1, Wc2, Wc3 = W2c[:50], W2c[50:100], W2c[100:]
    M2 = W2a @ W2b_top @ Wc1
    M1 = W2b_bot @ Wc1 + W2a @ Wc2
    M0 = Wc3
    v2 = (b2a @ W2b_top) @ Wc1
    v1 = b2b @ Wc1 + b2a @ Wc2
    cr = b2c + mcol[:, 0]
    cstT = jnp.concatenate(
        [M2.T, M1.T, M0.T, v2[:, None], v1[:, None], cr[:, None]], axis=1)

    outT = _combine(u1, u2, u3, xT, cstT)
    return outT[:, :N].T


# speedup vs baseline: 50.7233x; 1.3648x over previous
import functools

import jax
import jax.numpy as jnp
from jax import lax
from jax.experimental import pallas as pl
from jax.experimental.pallas import tpu as pltpu
from jax.experimental.pallas import tpu_sc as plsc

N = 100000
E = 6400000
CH = 128
NCH = E // CH
NW = 32
ROWS_PER_TILE = 6272
N_PAD = 16 * ROWS_PER_TILE
QUOT, REM = divmod(NCH, NW)
BLK_CH = 64
NBLOCKS = (QUOT + 1 + BLK_CH - 1) // BLK_CH

_mesh = plsc.VectorSubcoreMesh(core_axis_name="c", subcore_axis_name="s")


def _pass_body(u_in, src2d, dst2d, zeros1, out,
               a0, a1, a2, a3, u0, u1, u2, u3, sblk, dblk, vals, gsem, ssem):
    c = lax.axis_index("c")
    s = lax.axis_index("s")
    r0 = s * ROWS_PER_TILE
    accs = (a0, a1, a2, a3)
    usp = (u0, u1, u2, u3)
    for a in accs:
        pltpu.sync_copy(zeros1.at[pl.ds(r0, ROWS_PER_TILE)],
                        a.at[pl.ds(r0, ROWS_PER_TILE)])
    for k in range(4):
        pltpu.sync_copy(u_in.at[k].at[pl.ds(r0, ROWS_PER_TILE)],
                        usp[k].at[pl.ds(r0, ROWS_PER_TILE)])
    plsc.subcore_barrier()

    gwid = c * 16 + s
    nchunks = QUOT + (gwid < REM).astype(jnp.int32)
    base = gwid * QUOT + jnp.minimum(gwid, REM)

    dummy = zeros1.at[pl.ds(0, CH)]

    def block_body(b, carry):
        row0 = base + b * BLK_CH
        cnt = jnp.minimum(nchunks - b * BLK_CH, BLK_CH)
        pltpu.sync_copy(src2d.at[pl.ds(row0, BLK_CH)], sblk)
        pltpu.sync_copy(dst2d.at[pl.ds(row0, BLK_CH)], dblk)

        def chunk_body(j, carry2):
            g = b * BLK_CH + j
            par = lax.rem(g, 2)

            @pl.when(g >= 2)
            def _():
                for k in range(4):
                    pltpu.make_async_copy(dummy, vals.at[par].at[k],
                                          ssem).wait()
            descs = [pltpu.async_copy(usp[k].at[sblk.at[j]],
                                      vals.at[par].at[k], gsem)
                     for k in range(4)]
            for d in descs:
                d.wait()
            for k in range(4):
                pltpu.async_copy(vals.at[par].at[k], accs[k].at[dblk.at[j]],
                                 ssem, add=True)
            return carry2
        lax.fori_loop(0, cnt, chunk_body, 0)
        return carry
    lax.fori_loop(0, NBLOCKS, block_body, 0)
    for par in range(2):
        for k in range(4):
            pltpu.make_async_copy(dummy, vals.at[par].at[k], ssem).wait()
    plsc.subcore_barrier()
    for k, a in enumerate(accs):
        pltpu.sync_copy(a.at[pl.ds(r0, ROWS_PER_TILE)],
                        out.at[c, k, pl.ds(r0, ROWS_PER_TILE)])


_sc_pass = pl.kernel(
    _pass_body,
    out_type=jax.ShapeDtypeStruct((2, 4, N_PAD), jnp.float32),
    mesh=_mesh,
    scratch_types=[
        pltpu.VMEM_SHARED((N_PAD,), jnp.float32),
        pltpu.VMEM_SHARED((N_PAD,), jnp.float32),
        pltpu.VMEM_SHARED((N_PAD,), jnp.float32),
        pltpu.VMEM_SHARED((N_PAD,), jnp.float32),
        pltpu.VMEM_SHARED((N_PAD,), jnp.float32),
        pltpu.VMEM_SHARED((N_PAD,), jnp.float32),
        pltpu.VMEM_SHARED((N_PAD,), jnp.float32),
        pltpu.VMEM_SHARED((N_PAD,), jnp.float32),
        pltpu.VMEM((BLK_CH, CH), jnp.int32),
        pltpu.VMEM((BLK_CH, CH), jnp.int32),
        pltpu.VMEM((2, 4, CH), jnp.float32),
        pltpu.SemaphoreType.DMA,
        pltpu.SemaphoreType.DMA,
    ],
    compiler_params=pltpu.CompilerParams(use_tc_tiling_on_sc=False),
)

_PBLK = 3584


def _prologue_body(xT_ref, z_ref, m_ref):
    p = pl.program_id(0)
    i = pl.program_id(1)

    @pl.when((p == 0) & (i == 0))
    def _():
        m_ref[...] = jnp.zeros((3, 1), jnp.float32)

    @pl.when(p == 0)
    def _():
        m_ref[...] += jnp.sum(xT_ref[...], axis=1, keepdims=True) / N

    @pl.when(p == 1)
    def _():
        xc = xT_ref[...] - m_ref[...]
        z_ref[...] = jnp.concatenate(
            [xc, jnp.ones((1, _PBLK), jnp.float32)], axis=0)


def _prologue(xT):
    return pl.pallas_call(
        _prologue_body,
        grid=(2, N_PAD // _PBLK),
        in_specs=[pl.BlockSpec((3, _PBLK), lambda p, i: (0, i))],
        out_specs=[pl.BlockSpec((4, _PBLK), lambda p, i: (0, i)),
                   pl.BlockSpec((3, 1), lambda p, i: (0, 0))],
        out_shape=[jax.ShapeDtypeStruct((4, N_PAD), jnp.float32),
                   jax.ShapeDtypeStruct((3, 1), jnp.float32)],
    )(xT)


def _add_body(p_ref, o_ref):
    o_ref[...] = p_ref[0] + p_ref[1]


def _tc_add(p):
    return pl.pallas_call(
        _add_body,
        grid=(N_PAD // _PBLK,),
        in_specs=[pl.BlockSpec((2, 4, _PBLK), lambda i: (0, 0, i))],
        out_specs=pl.BlockSpec((4, _PBLK), lambda i: (0, i)),
        out_shape=jax.ShapeDtypeStruct((4, N_PAD), jnp.float32),
    )(p)


_BLK = 3584


def _combine_body(u1_ref, u2_ref, u3_ref, xT_ref, cst_ref, o_ref):
    pid = pl.program_id(0)
    u1 = u1_ref[...]
    u2 = u2_ref[...]
    u3 = u3_ref[...]
    cst = cst_ref[...]
    M2T, M1T, M0T = cst[:, 0:3], cst[:, 3:6], cst[:, 6:9]
    v2c, v1c, crc = cst[:, 9:10], cst[:, 10:11], cst[:, 11:12]
    x3 = (jnp.dot(M2T, u3[:3]) + jnp.dot(M1T, u2[:3])
          + jnp.dot(M0T, u1[:3])
          + v2c * u2[3:4] + v1c * u1[3:4] + crc)
    r = pid * _BLK + lax.broadcasted_iota(jnp.int32, (1, _BLK), 1)
    rm = r % 40
    fixed = (r < 1960) & ((rm <= 13) | ((rm >= 25) & (rm <= 38)))
    o_ref[...] = jnp.where(fixed, xT_ref[...], x3)


def _combine(u1, u2, u3, xT, cstT):
    return pl.pallas_call(
        _combine_body,
        grid=(N_PAD // _BLK,),
        in_specs=[
            pl.BlockSpec((4, _BLK), lambda i: (0, i)),
            pl.BlockSpec((4, _BLK), lambda i: (0, i)),
            pl.BlockSpec((4, _BLK), lambda i: (0, i)),
            pl.BlockSpec((3, _BLK), lambda i: (0, i)),
            pl.BlockSpec((3, 12), lambda i: (0, 0)),
        ],
        out_specs=pl.BlockSpec((3, _BLK), lambda i: (0, i)),
        out_shape=jax.ShapeDtypeStruct((3, N_PAD), jnp.float32),
    )(u1, u2, u3, xT, cstT)


def kernel(x, edge_index, edge_weight,
           W1a, b1a, W2a, b2a, W1b, b1b, W2b, b2b, W1c, b1c, W2c, b2c):
    pad = jnp.zeros((BLK_CH, CH), jnp.int32)
    src2d = jnp.concatenate([edge_index[0].reshape(NCH, CH), pad], axis=0)
    dst2d = jnp.concatenate([edge_index[1].reshape(NCH, CH), pad], axis=0)
    zeros1 = jnp.zeros((N_PAD,), jnp.float32)

    xT = jnp.concatenate(
        [x.T, jnp.zeros((3, N_PAD - N), jnp.float32)], axis=1)
    z, mcol = _prologue(xT)

    u1 = _tc_add(_sc_pass(z, src2d, dst2d, zeros1))
    u2 = _tc_add(_sc_pass(u1, src2d, dst2d, zeros1))
    u3 = _tc_add(_sc_pass(u2, src2d, dst2d, zeros1))

    W2b_top, W2b_bot = W2b[:50], W2b[50:]
    Wc1, Wc2, Wc3 = W2c[:50], W2c[50:100], W2c[100:]
    M2 = W2a @ W2b_top @ Wc1
    M1 = W2b_bot @ Wc1 + W2a @ Wc2
    M0 = Wc3
    v2 = (b2a @ W2b_top) @ Wc1
    v1 = b2b @ Wc1 + b2a @ Wc2
    cr = b2c + mcol[:, 0]
    cstT = jnp.concatenate(
        [M2.T, M1.T, M0.T, v2[:, None], v1[:, None], cr[:, None]], axis=1)

    outT = _combine(u1, u2, u3, xT, cstT)
    return outT[:, :N].T


# speedup vs baseline: 52.7146x; 1.0393x over previous
import functools

import jax
import jax.numpy as jnp
from jax import lax
from jax.experimental import pallas as pl
from jax.experimental.pallas import tpu as pltpu
from jax.experimental.pallas import tpu_sc as plsc

N = 100000
E = 6400000
CH = 128
NCH = E // CH
NW = 32
ROWS_PER_TILE = 6272
N_PAD = 16 * ROWS_PER_TILE
QUOT, REM = divmod(NCH, NW)
BLK_CH = 64
NBLOCKS = (QUOT + 1 + BLK_CH - 1) // BLK_CH

_mesh = plsc.VectorSubcoreMesh(core_axis_name="c", subcore_axis_name="s")


def _pass_body(first, u_in, src2d, dst2d, zeros1, ones1, out,
               a0, a1, a2, a3, u0, u1, u2, u3, sblk, dblk, vals, onesb,
               gsem, ssem):
    c = lax.axis_index("c")
    s = lax.axis_index("s")
    r0 = s * ROWS_PER_TILE
    accs = (a0, a1, a2, a3)
    usp = (u0, u1, u2, u3)
    for a in accs:
        pltpu.sync_copy(zeros1.at[pl.ds(r0, ROWS_PER_TILE)],
                        a.at[pl.ds(r0, ROWS_PER_TILE)])
    nk = 3 if first else 4
    for k in range(nk):
        pltpu.sync_copy(u_in.at[k].at[pl.ds(r0, ROWS_PER_TILE)],
                        usp[k].at[pl.ds(r0, ROWS_PER_TILE)])
    if first:
        pltpu.sync_copy(ones1.at[pl.ds(0, CH)], onesb)
    plsc.subcore_barrier()

    gwid = c * 16 + s
    nchunks = QUOT + (gwid < REM).astype(jnp.int32)
    base = gwid * QUOT + jnp.minimum(gwid, REM)

    dummy = zeros1.at[pl.ds(0, CH)]

    def block_body(b, carry):
        row0 = base + b * BLK_CH
        cnt = jnp.minimum(nchunks - b * BLK_CH, BLK_CH)
        pltpu.sync_copy(src2d.at[pl.ds(row0, BLK_CH)], sblk)
        pltpu.sync_copy(dst2d.at[pl.ds(row0, BLK_CH)], dblk)

        def chunk_body(j, carry2):
            g = b * BLK_CH + j
            par = lax.rem(g, 2)

            @pl.when(g >= 2)
            def _():
                for k in range(nk):
                    pltpu.make_async_copy(dummy, vals.at[par].at[k],
                                          ssem).wait()
                if first:
                    pltpu.make_async_copy(dummy, onesb, ssem).wait()
            descs = [pltpu.async_copy(usp[k].at[sblk.at[j]],
                                      vals.at[par].at[k], gsem)
                     for k in range(nk)]
            for d in descs:
                d.wait()
            for k in range(nk):
                pltpu.async_copy(vals.at[par].at[k], accs[k].at[dblk.at[j]],
                                 ssem, add=True)
            if first:
                pltpu.async_copy(onesb, accs[3].at[dblk.at[j]],
                                 ssem, add=True)
            return carry2
        lax.fori_loop(0, cnt, chunk_body, 0)
        return carry
    lax.fori_loop(0, NBLOCKS, block_body, 0)
    for par in range(2):
        for k in range(nk):
            pltpu.make_async_copy(dummy, vals.at[par].at[k], ssem).wait()
        if first:
            pltpu.make_async_copy(dummy, onesb, ssem).wait()
    plsc.subcore_barrier()
    for k, a in enumerate(accs):
        pltpu.sync_copy(a.at[pl.ds(r0, ROWS_PER_TILE)],
                        out.at[c, k, pl.ds(r0, ROWS_PER_TILE)])


def _make_pass(first):
    return pl.kernel(
        functools.partial(_pass_body, first),
        out_type=jax.ShapeDtypeStruct((2, 4, N_PAD), jnp.float32),
        mesh=_mesh,
        scratch_types=[
            pltpu.VMEM_SHARED((N_PAD,), jnp.float32),
            pltpu.VMEM_SHARED((N_PAD,), jnp.float32),
            pltpu.VMEM_SHARED((N_PAD,), jnp.float32),
            pltpu.VMEM_SHARED((N_PAD,), jnp.float32),
            pltpu.VMEM_SHARED((N_PAD,), jnp.float32),
            pltpu.VMEM_SHARED((N_PAD,), jnp.float32),
            pltpu.VMEM_SHARED((N_PAD,), jnp.float32),
            pltpu.VMEM_SHARED((N_PAD,), jnp.float32),
            pltpu.VMEM((BLK_CH, CH), jnp.int32),
            pltpu.VMEM((BLK_CH, CH), jnp.int32),
            pltpu.VMEM((2, 4, CH), jnp.float32),
            pltpu.VMEM((CH,), jnp.float32),
            pltpu.SemaphoreType.DMA,
            pltpu.SemaphoreType.DMA,
        ],
        compiler_params=pltpu.CompilerParams(use_tc_tiling_on_sc=False),
    )


_pass_first = _make_pass(True)
_pass_next = _make_pass(False)


_PBLK = 3584


def _prologue_body(xT_ref, z_ref, m_ref):
    p = pl.program_id(0)
    i = pl.program_id(1)

    @pl.when((p == 0) & (i == 0))
    def _():
        m_ref[...] = jnp.zeros((3, 1), jnp.float32)

    @pl.when(p == 0)
    def _():
        m_ref[...] += jnp.sum(xT_ref[...], axis=1, keepdims=True) / N

    @pl.when(p == 1)
    def _():
        xc = xT_ref[...] - m_ref[...]
        z_ref[...] = jnp.concatenate(
            [xc, jnp.ones((1, _PBLK), jnp.float32)], axis=0)


def _prologue(xT):
    return pl.pallas_call(
        _prologue_body,
        grid=(2, N_PAD // _PBLK),
        in_specs=[pl.BlockSpec((3, _PBLK), lambda p, i: (0, i))],
        out_specs=[pl.BlockSpec((4, _PBLK), lambda p, i: (0, i)),
                   pl.BlockSpec((3, 1), lambda p, i: (0, 0))],
        out_shape=[jax.ShapeDtypeStruct((4, N_PAD), jnp.float32),
                   jax.ShapeDtypeStruct((3, 1), jnp.float32)],
    )(xT)


def _add_body(p_ref, o_ref):
    o_ref[...] = p_ref[0] + p_ref[1]


def _tc_add(p):
    return pl.pallas_call(
        _add_body,
        grid=(N_PAD // _PBLK,),
        in_specs=[pl.BlockSpec((2, 4, _PBLK), lambda i: (0, 0, i))],
        out_specs=pl.BlockSpec((4, _PBLK), lambda i: (0, i)),
        out_shape=jax.ShapeDtypeStruct((4, N_PAD), jnp.float32),
    )(p)


_BLK = 3584


def _combine_body(u1_ref, u2_ref, u3_ref, xT_ref, cst_ref, o_ref):
    pid = pl.program_id(0)
    u1 = u1_ref[...]
    u2 = u2_ref[...]
    u3 = u3_ref[...]
    cst = cst_ref[...]
    M2T, M1T, M0T = cst[:, 0:3], cst[:, 3:6], cst[:, 6:9]
    v2c, v1c, crc = cst[:, 9:10], cst[:, 10:11], cst[:, 11:12]
    x3 = (jnp.dot(M2T, u3[:3]) + jnp.dot(M1T, u2[:3])
          + jnp.dot(M0T, u1[:3])
          + v2c * u2[3:4] + v1c * u1[3:4] + crc)
    r = pid * _BLK + lax.broadcasted_iota(jnp.int32, (1, _BLK), 1)
    rm = r % 40
    fixed = (r < 1960) & ((rm <= 13) | ((rm >= 25) & (rm <= 38)))
    o_ref[...] = jnp.where(fixed, xT_ref[...], x3)


def _combine(u1, u2, u3, xT, cstT):
    return pl.pallas_call(
        _combine_body,
        grid=(N_PAD // _BLK,),
        in_specs=[
            pl.BlockSpec((4, _BLK), lambda i: (0, i)),
            pl.BlockSpec((4, _BLK), lambda i: (0, i)),
            pl.BlockSpec((4, _BLK), lambda i: (0, i)),
            pl.BlockSpec((3, _BLK), lambda i: (0, i)),
            pl.BlockSpec((3, 12), lambda i: (0, 0)),
        ],
        out_specs=pl.BlockSpec((3, _BLK), lambda i: (0, i)),
        out_shape=jax.ShapeDtypeStruct((3, N_PAD), jnp.float32),
    )(u1, u2, u3, xT, cstT)


def kernel(x, edge_index, edge_weight,
           W1a, b1a, W2a, b2a, W1b, b1b, W2b, b2b, W1c, b1c, W2c, b2c):
    pad = jnp.zeros((BLK_CH, CH), jnp.int32)
    src2d = jnp.concatenate([edge_index[0].reshape(NCH, CH), pad], axis=0)
    dst2d = jnp.concatenate([edge_index[1].reshape(NCH, CH), pad], axis=0)
    zeros1 = jnp.zeros((N_PAD,), jnp.float32)
    ones1 = jnp.ones((CH,), jnp.float32)

    xT = jnp.concatenate(
        [x.T, jnp.zeros((3, N_PAD - N), jnp.float32)], axis=1)
    z, mcol = _prologue(xT)

    u1 = _tc_add(_pass_first(z, src2d, dst2d, zeros1, ones1))
    u2 = _tc_add(_pass_next(u1, src2d, dst2d, zeros1, ones1))
    u3 = _tc_add(_pass_next(u2, src2d, dst2d, zeros1, ones1))

    W2b_top, W2b_bot = W2b[:50], W2b[50:]
    Wc1, Wc2, Wc3 = W2c[:50], W2c[50:100], W2c[100:]
    M2 = W2a @ W2b_top @ Wc1
    M1 = W2b_bot @ Wc1 + W2a @ Wc2
    M0 = Wc3
    v2 = (b2a @ W2b_top) @ Wc1
    v1 = b2b @ Wc1 + b2a @ Wc2
    cr = b2c + mcol[:, 0]
    cstT = jnp.concatenate(
        [M2.T, M1.T, M0.T, v2[:, None], v1[:, None], cr[:, None]], axis=1)

    outT = _combine(u1, u2, u3, xT, cstT)
    return outT[:, :N].T
